# baseline (speedup 1.0000x reference)
import jax
import jax.numpy as jnp
from jax.experimental import pallas as pl
from jax.experimental.pallas import tpu as pltpu

_R = 512


def kernel(sequence, table):
    batch, hist = sequence.shape
    vocab, embed = table.shape
    n = batch * hist
    nblocks = n // _R
    idx = sequence.reshape(nblocks, 1, _R)

    def body(idx_ref, table_ref, out_ref):
        def row(r, carry):
            i = idx_ref[0, 0, r]
            out_ref[pl.ds(r, 1), :] = table_ref[pl.ds(i, 1), :]
            return carry

        jax.lax.fori_loop(0, _R, row, 0, unroll=8)

    out = pl.pallas_call(
        body,
        grid=(nblocks,),
        in_specs=[
            pl.BlockSpec((1, 1, _R), lambda i: (i, 0, 0), memory_space=pltpu.SMEM),
            pl.BlockSpec((vocab, embed), lambda i: (0, 0)),
        ],
        out_specs=pl.BlockSpec((_R, embed), lambda i: (i, 0)),
        out_shape=jax.ShapeDtypeStruct((n, embed), table.dtype),
        compiler_params=pltpu.CompilerParams(
            dimension_semantics=("arbitrary",),
        ),
    )(idx, table)
    return out.reshape(batch, hist, embed)
